# baseline (speedup 1.0000x reference)
import functools

import jax
import jax.numpy as jnp
from jax import lax
from jax.experimental import pallas as pl
from jax.experimental.pallas import tpu as pltpu
from jax.experimental.pallas import tpu_sc as plsc

N = 10000
E = 320000
D = 128
DH = 64
NC = 2
NS = 16
EPT = E // NS
B = 80
NB = EPT // B
NPAIR = NB // 2
EPR = 624
TAIL = N - NS * EPR
RZ = 208
NPD = 10240
DPT = NPD // NS
E2 = E // NC
WPT = E2 // NS
WB_ = WPT // B
NBALL = E // B

_mesh = plsc.VectorSubcoreMesh(core_axis_name="c", subcore_axis_name="s")


GB = 50
NG = NB // GB
PPG = GB // 2


def _scale_rows(rows, wv, j):
    def body(g, carry):
        wvec = wv[j, pl.ds(g * 16, 16)]
        for u in range(16):
            e = g * 16 + u
            we = wvec[u]
            for k in range(DH // 16):
                rows[e, pl.ds(k * 16, 16)] = rows[e, pl.ds(k * 16, 16)] * we
        return carry
    lax.fori_loop(0, B // 16, body, 0)


def _prop_stage(tblref, srcb, dstb, wb, s, coff, gidx, didx, wv,
                rows0, rows1, acc, gsem0, gsem1, ssem0, ssem1):
    def group(g, gcarry):
        base = s * NB + g * GB
        pltpu.sync_copy(srcb.at[pl.ds(base, GB)], gidx)
        pltpu.sync_copy(dstb.at[pl.ds(base, GB)], didx)
        pltpu.sync_copy(wb.at[pl.ds(base, GB)], wv)

        def _adj(j, carry):
            for k in range(B // 16):
                gidx[j, pl.ds(k * 16, 16)] = gidx[j, pl.ds(k * 16, 16)] + coff
            return carry
        lax.fori_loop(0, GB, _adj, 0)

        pltpu.async_copy(tblref.at[gidx.at[0]], rows0, gsem0)

        def body(p, carry):
            j0 = 2 * p
            j1 = j0 + 1

            @pl.when(p > 0)
            def _():
                pltpu.make_async_copy(rows1, acc.at[didx.at[0]], ssem1).wait()

            pltpu.async_copy(tblref.at[gidx.at[j1]], rows1, gsem1)
            pltpu.make_async_copy(tblref.at[gidx.at[j0]], rows0, gsem0).wait()
            _scale_rows(rows0, wv, j0)
            pltpu.async_copy(rows0, acc.at[didx.at[j0]], ssem0, add=True)
            pltpu.make_async_copy(tblref.at[gidx.at[j1]], rows1, gsem1).wait()
            _scale_rows(rows1, wv, j1)
            pltpu.async_copy(rows1, acc.at[didx.at[j1]], ssem1, add=True)
            pltpu.make_async_copy(rows0, acc.at[didx.at[0]], ssem0).wait()

            @pl.when(p < PPG - 1)
            def _():
                pltpu.async_copy(tblref.at[gidx.at[j0 + 2]], rows0, gsem0)

            return carry

        lax.fori_loop(0, PPG, body, 0)
        pltpu.make_async_copy(rows1, acc.at[didx.at[0]], ssem1).wait()
        return gcarry

    lax.fori_loop(0, NG, group, 0)


@functools.partial(
    pl.kernel,
    out_type=[
        jax.ShapeDtypeStruct((2 * N, DH), jnp.float32),
        jax.ShapeDtypeStruct((2 * N, DH), jnp.float32),
    ],
    mesh=_mesh,
    scratch_types=[
        pltpu.VMEM((GB, B), jnp.int32),
        pltpu.VMEM((GB, B), jnp.int32),
        pltpu.VMEM((GB, B), jnp.float32),
        pltpu.VMEM((B, DH), jnp.float32),
        pltpu.VMEM((B, DH), jnp.float32),
        pltpu.VMEM_SHARED((N, DH), jnp.float32),
        pltpu.VMEM_SHARED((N, DH), jnp.float32),
        pltpu.SemaphoreType.DMA,
        pltpu.SemaphoreType.DMA,
        pltpu.SemaphoreType.DMA,
        pltpu.SemaphoreType.DMA,
        pltpu.SemaphoreType.DMA,
    ],
    compiler_params=pltpu.CompilerParams(needs_layout_passes=False,
                                         use_tc_tiling_on_sc=False),
)
def _pair_prop(tbl, srcb, dstb, wb, p1_out, p2_out,
               gidx, didx, wv, rows0, rows1, acc1, acc2,
               gsem0, gsem1, ssem0, ssem1, epsem):
    c = lax.axis_index("c")
    s = lax.axis_index("s")

    zv = jnp.zeros((16,), jnp.float32)

    def _zr(i, carry):
        for k in range(DH // 16):
            rows0[i, pl.ds(k * 16, 16)] = zv
        return carry
    lax.fori_loop(0, B, _zr, 0)
    for p in range(EPR // B):
        pltpu.sync_copy(rows0, acc1.at[pl.ds(s * EPR + p * B, B)])
        pltpu.sync_copy(rows0, acc2.at[pl.ds(s * EPR + p * B, B)])
    rem = EPR - (EPR // B) * B
    if rem:
        pltpu.sync_copy(rows0.at[pl.ds(0, rem)],
                        acc1.at[pl.ds(s * EPR + (EPR // B) * B, rem)])
        pltpu.sync_copy(rows0.at[pl.ds(0, rem)],
                        acc2.at[pl.ds(s * EPR + (EPR // B) * B, rem)])

    @pl.when(s == NS - 1)
    def _():
        pltpu.sync_copy(rows0.at[pl.ds(0, TAIL)], acc1.at[pl.ds(NS * EPR, TAIL)])
        pltpu.sync_copy(rows0.at[pl.ds(0, TAIL)], acc2.at[pl.ds(NS * EPR, TAIL)])

    coff = c * N

    plsc.subcore_barrier()
    _prop_stage(tbl, srcb, dstb, wb, s, coff, gidx, didx, wv,
                rows0, rows1, acc1, gsem0, gsem1, ssem0, ssem1)
    plsc.subcore_barrier()
    ep1 = pltpu.async_copy(acc1.at[pl.ds(s * EPR, EPR)],
                           p1_out.at[pl.ds(c * N + s * EPR, EPR)], epsem)

    @pl.when(s == NS - 1)
    def _():
        pltpu.sync_copy(acc1.at[pl.ds(NS * EPR, TAIL)],
                        p1_out.at[pl.ds(c * N + NS * EPR, TAIL)])

    ep1.wait()
    plsc.subcore_barrier()
    _prop_stage(p1_out, srcb, dstb, wb, s, coff, gidx, didx, wv,
                rows0, rows1, acc2, gsem0, gsem1, ssem0, ssem1)
    plsc.subcore_barrier()
    pltpu.sync_copy(acc2.at[pl.ds(s * EPR, EPR)],
                    p2_out.at[pl.ds(c * N + s * EPR, EPR)])

    @pl.when(s == NS - 1)
    def _():
        pltpu.sync_copy(acc2.at[pl.ds(NS * EPR, TAIL)],
                        p2_out.at[pl.ds(c * N + NS * EPR, TAIL)])


@functools.partial(
    pl.kernel,
    out_type=jax.ShapeDtypeStruct((NBALL, B), jnp.float32),
    mesh=_mesh,
    scratch_types=[
        pltpu.VMEM((NB, B), jnp.int32),
        pltpu.VMEM((NB, B), jnp.float32),
        pltpu.VMEM((B, 16), jnp.float32),
        pltpu.VMEM((B, 16), jnp.float32),
        pltpu.VMEM((DPT, 16), jnp.float32),
        pltpu.VMEM((DPT,), jnp.float32),
        pltpu.VMEM((NPD,), jnp.float32),
        pltpu.VMEM((WB_, B), jnp.int32),
        pltpu.VMEM((WB_, B), jnp.int32),
        pltpu.VMEM((WB_, B), jnp.float32),
        pltpu.VMEM((WB_, B), jnp.float32),
        pltpu.VMEM_SHARED((NPD, 16), jnp.float32),
        pltpu.VMEM_SHARED((NPD,), jnp.float32),
        pltpu.SemaphoreType.DMA,
        pltpu.SemaphoreType.DMA,
    ],
    compiler_params=pltpu.CompilerParams(needs_layout_passes=False,
                                         use_tc_tiling_on_sc=False),
)
def _edge_prep(srcb, dstb, ewb, w_out,
               sidx, ewv, rows0d, rows1d, deg16, disbuf, disfull,
               srcw, dstw, eww, wout, deg_acc, dis_sh, ssem0, ssem1):
    c = lax.axis_index("c")
    s = lax.axis_index("s")
    zv = jnp.zeros((16,), jnp.float32)

    def _z(i, carry):
        rows0d[i, pl.ds(0, 16)] = zv
        return carry
    lax.fori_loop(0, B, _z, 0)
    for k in range(DPT // B):
        pltpu.sync_copy(rows0d, deg_acc.at[pl.ds(s * DPT + k * B, B)])
    pltpu.sync_copy(srcb.at[pl.ds(s * NB, NB)], sidx)
    pltpu.sync_copy(ewb.at[pl.ds(s * NB, NB)], ewv)
    plsc.subcore_barrier()

    def _fill(rows, j):
        def fbody(g, carry):
            ev = ewv[j, pl.ds(g * 16, 16)]
            for u in range(16):
                rows[g * 16 + u, pl.ds(0, 16)] = ev[u] + zv
            return carry
        lax.fori_loop(0, B // 16, fbody, 0)

    def _dbody(p, carry):
        j0 = 2 * p
        j1 = j0 + 1
        _fill(rows0d, j0)
        pltpu.async_copy(rows0d, deg_acc.at[sidx.at[j0]], ssem0, add=True)

        @pl.when(p > 0)
        def _():
            pltpu.make_async_copy(rows1d, deg_acc.at[sidx.at[0]], ssem1).wait()

        _fill(rows1d, j1)
        pltpu.async_copy(rows1d, deg_acc.at[sidx.at[j1]], ssem1, add=True)
        pltpu.make_async_copy(rows0d, deg_acc.at[sidx.at[0]], ssem0).wait()
        return carry
    lax.fori_loop(0, NPAIR, _dbody, 0)
    pltpu.make_async_copy(rows1d, deg_acc.at[sidx.at[0]], ssem1).wait()
    plsc.subcore_barrier()

    pltpu.sync_copy(deg_acc.at[pl.ds(s * DPT, DPT)], deg16)
    czero = jnp.zeros((16,), jnp.int32)

    def _dis(i, carry):
        ridx = lax.iota(jnp.int32, 16) + i * 16
        d = plsc.load_gather(deg16, [ridx, czero])
        m = d > 0.0
        d1 = jnp.where(m, d, 1.0)
        ii = plsc.bitcast(d1, jnp.int32)
        ii = 0x5F3759DF - jnp.right_shift(ii, 1)
        y = plsc.bitcast(ii, jnp.float32)
        for _it in range(3):
            y = y * (1.5 - 0.5 * d1 * y * y)
        disbuf[pl.ds(i * 16, 16)] = jnp.where(m, y, 0.0)
        return carry
    lax.fori_loop(0, DPT // 16, _dis, 0)
    pltpu.sync_copy(disbuf, dis_sh.at[pl.ds(s * DPT, DPT)])
    plsc.subcore_barrier()
    pltpu.sync_copy(dis_sh, disfull)

    wbase = c * (E2 // B) + s * WB_
    pltpu.sync_copy(srcb.at[pl.ds(wbase, WB_)], srcw)
    pltpu.sync_copy(dstb.at[pl.ds(wbase, WB_)], dstw)
    pltpu.sync_copy(ewb.at[pl.ds(wbase, WB_)], eww)

    def _w(j, carry):
        for k in range(B // 16):
            sv = srcw[j, pl.ds(k * 16, 16)]
            dv = dstw[j, pl.ds(k * 16, 16)]
            ev = eww[j, pl.ds(k * 16, 16)]
            a = plsc.load_gather(disfull, [sv])
            bb = plsc.load_gather(disfull, [dv])
            wout[j, pl.ds(k * 16, 16)] = -(a * ev * bb)
        return carry
    lax.fori_loop(0, WB_, _w, 0)
    pltpu.sync_copy(wout, w_out.at[pl.ds(wbase, WB_)])


BN = 1000
NBLK = N // BN


def _dot(a, b):
    return lax.dot_general(a, b, (((1,), (0,)), ((), ())),
                           precision=lax.Precision.HIGHEST,
                           preferred_element_type=jnp.float32)


def _tc1_body(x, h, p1xl, p1xh, p2xl, p2xh, p1hl, p1hh, p2hl, p2hh,
              ax, a1, a2, bh, b1, b2, bias, z_ref, ht1_ref, hrs_ref):
    a1v = a1[...]
    a2v = a2[...]
    b1v = b1[...]
    b2v = b2[...]
    accx = (_dot(x[...], ax[...])
            + _dot(p1xl[...], a1v[:DH]) + _dot(p1xh[...], a1v[DH:])
            + _dot(p2xl[...], a2v[:DH]) + _dot(p2xh[...], a2v[DH:]))
    acch = (_dot(h[...], bh[...])
            + _dot(p1hl[...], b1v[:DH]) + _dot(p1hh[...], b1v[DH:])
            + _dot(p2hl[...], b2v[:DH]) + _dot(p2hh[...], b2v[DH:]))
    pre = accx + bias[...]
    z = jax.nn.sigmoid(pre[:, :D] + acch[:, :D])
    r = jax.nn.sigmoid(pre[:, D:2 * D] + acch[:, D:])
    z_ref[...] = z
    ht1_ref[...] = pre[:, 2 * D:]
    hr = h[...] * r
    hrs_ref[0] = hr[:, :DH]
    hrs_ref[1] = hr[:, DH:]


def _tc2_body(hrl, hrh, p1l, p1h_, p2l, p2h_, ht1, z, h,
              c0, c1, c2, bias, out_ref):
    c0v = c0[...]
    c1v = c1[...]
    c2v = c2[...]
    ht2 = (_dot(hrl[...], c0v[:DH]) + _dot(hrh[...], c0v[DH:])
           + _dot(p1l[...], c1v[:DH]) + _dot(p1h_[...], c1v[DH:])
           + _dot(p2l[...], c2v[:DH]) + _dot(p2h_[...], c2v[DH:])
           + bias[...])
    sacc = ht1[...] + ht2
    htil = jnp.tanh(sacc) + sacc
    zv = z[...]
    out_ref[...] = zv * h[...] + (1.0 - zv) * htil


def _rb(i):
    return (i, 0)


def _hb(i):
    return (i + NBLK, 0)


def _full(i):
    return (0, 0)


def _tc1(X, H, P1x, P2x, P1h, P2h, Ax, A1, A2, Bh, B1, B2, bias):
    return pl.pallas_call(
        _tc1_body,
        grid=(NBLK,),
        in_specs=[
            pl.BlockSpec((BN, D), _rb),
            pl.BlockSpec((BN, D), _rb),
            pl.BlockSpec((BN, DH), _rb), pl.BlockSpec((BN, DH), _hb),
            pl.BlockSpec((BN, DH), _rb), pl.BlockSpec((BN, DH), _hb),
            pl.BlockSpec((BN, DH), _rb), pl.BlockSpec((BN, DH), _hb),
            pl.BlockSpec((BN, DH), _rb), pl.BlockSpec((BN, DH), _hb),
            pl.BlockSpec((D, 3 * D), _full), pl.BlockSpec((D, 3 * D), _full),
            pl.BlockSpec((D, 3 * D), _full),
            pl.BlockSpec((D, 2 * D), _full), pl.BlockSpec((D, 2 * D), _full),
            pl.BlockSpec((D, 2 * D), _full),
            pl.BlockSpec((1, 3 * D), _full),
        ],
        out_specs=[
            pl.BlockSpec((BN, D), _rb),
            pl.BlockSpec((BN, D), _rb),
            pl.BlockSpec((2, BN, DH), lambda i: (0, i, 0)),
        ],
        out_shape=[
            jax.ShapeDtypeStruct((N, D), jnp.float32),
            jax.ShapeDtypeStruct((N, D), jnp.float32),
            jax.ShapeDtypeStruct((2, N, DH), jnp.float32),
        ],
    )(X, H, P1x, P1x, P2x, P2x, P1h, P1h, P2h, P2h,
      Ax, A1, A2, Bh, B1, B2, bias)


def _tc2(HRs, P1hr, P2hr, Ht1, Z, H, C0, C1, C2, bias):
    return pl.pallas_call(
        _tc2_body,
        grid=(NBLK,),
        in_specs=[
            pl.BlockSpec((BN, DH), _rb), pl.BlockSpec((BN, DH), _hb),
            pl.BlockSpec((BN, DH), _rb), pl.BlockSpec((BN, DH), _hb),
            pl.BlockSpec((BN, DH), _rb), pl.BlockSpec((BN, DH), _hb),
            pl.BlockSpec((BN, D), _rb),
            pl.BlockSpec((BN, D), _rb),
            pl.BlockSpec((BN, D), _rb),
            pl.BlockSpec((D, D), _full), pl.BlockSpec((D, D), _full),
            pl.BlockSpec((D, D), _full),
            pl.BlockSpec((1, D), _full),
        ],
        out_specs=pl.BlockSpec((BN, D), _rb),
        out_shape=jax.ShapeDtypeStruct((N, D), jnp.float32),
    )(HRs, HRs, P1hr, P1hr, P2hr, P2hr, Ht1, Z, H, C0, C1, C2, bias)


def kernel(X, edge_index, edge_weight, H,
           W_xz, b_xz, W_hz, b_hz, W_xr, b_xr, W_hr, b_hr,
           W_xh, b_xh, W_hh, b_hh):
    src = edge_index[0]
    dst = edge_index[1]
    srcb = src.reshape(NBALL, B)
    dstb = dst.reshape(NBALL, B)
    ewb = edge_weight.reshape(NBALL, B)

    wb = _edge_prep(srcb, dstb, ewb)

    Xs = jnp.concatenate([X[:, :DH], X[:, DH:]], axis=0)
    Hs = jnp.concatenate([H[:, :DH], H[:, DH:]], axis=0)
    P1x, P2x = _pair_prop(Xs, srcb, dstb, wb)
    P1h, P2h = _pair_prop(Hs, srcb, dstb, wb)

    Ax = jnp.concatenate([W_xz[0] - W_xz[2], W_xr[0] - W_xr[2],
                          W_xh[0] - W_xh[2]], axis=1)
    A1 = jnp.concatenate([W_xz[1], W_xr[1], W_xh[1]], axis=1)
    A2 = jnp.concatenate([2.0 * W_xz[2], 2.0 * W_xr[2], 2.0 * W_xh[2]], axis=1)
    Bh = jnp.concatenate([W_hz[0] - W_hz[2], W_hr[0] - W_hr[2]], axis=1)
    B1 = jnp.concatenate([W_hz[1], W_hr[1]], axis=1)
    B2 = jnp.concatenate([2.0 * W_hz[2], 2.0 * W_hr[2]], axis=1)
    bias = jnp.concatenate([b_xz + b_hz, b_xr + b_hr, b_xh])[None, :]

    Z, Ht1, HRs3 = _tc1(X, H, P1x, P2x, P1h, P2h, Ax, A1, A2, Bh, B1, B2, bias)
    HRs = HRs3.reshape(2 * N, DH)

    P1hr, P2hr = _pair_prop(HRs, srcb, dstb, wb)

    C0 = W_hh[0] - W_hh[2]
    C1 = W_hh[1]
    C2 = 2.0 * W_hh[2]
    return _tc2(HRs, P1hr, P2hr, Ht1, Z, H, C0, C1, C2, b_hh[None, :])


# speedup vs baseline: 1.1492x; 1.1492x over previous
import functools

import jax
import jax.numpy as jnp
from jax import lax
from jax.experimental import pallas as pl
from jax.experimental.pallas import tpu as pltpu
from jax.experimental.pallas import tpu_sc as plsc

N = 10000
E = 320000
D = 128
DH = 64
NC = 2
NS = 16
EPT = E // NS
B = 80
NB = EPT // B
NPAIR = NB // 2
EPR = 624
TAIL = N - NS * EPR
RZ = 208
NPD = 10240
DPT = NPD // NS
E2 = E // NC
WPT = E2 // NS
WB_ = WPT // B
NBALL = E // B

_mesh = plsc.VectorSubcoreMesh(core_axis_name="c", subcore_axis_name="s")


GB = 50
NG = NB // GB
PPG = GB // 2


def _scale_rows(rows, wv, j):
    def body(g, carry):
        wvec = wv[j, pl.ds(g * 16, 16)]
        for u in range(16):
            e = g * 16 + u
            we = wvec[u]
            for k in range(DH // 16):
                rows[e, pl.ds(k * 16, 16)] = rows[e, pl.ds(k * 16, 16)] * we
        return carry
    lax.fori_loop(0, B // 16, body, 0)


NRING = 5
RPG = GB // NRING


def _prop_stage(tblref, srcb, dstb, wb, s, coff, gidx, didx, wv,
                rows, acc, gsems, ssems):
    def group(g, gcarry):
        base = s * NB + g * GB
        pltpu.sync_copy(srcb.at[pl.ds(base, GB)], gidx)
        pltpu.sync_copy(dstb.at[pl.ds(base, GB)], didx)
        pltpu.sync_copy(wb.at[pl.ds(base, GB)], wv)

        def _adj(j, carry):
            for k in range(B // 16):
                gidx[j, pl.ds(k * 16, 16)] = gidx[j, pl.ds(k * 16, 16)] + coff
            return carry
        lax.fori_loop(0, GB, _adj, 0)

        def rnd(r, carry):
            jb = r * NRING
            for u in range(NRING):
                @pl.when(r > 0)
                def _():
                    pltpu.make_async_copy(rows[u], acc.at[didx.at[0]],
                                          ssems[u]).wait()
                pltpu.async_copy(tblref.at[gidx.at[jb + u]], rows[u], gsems[u])
            for u in range(NRING):
                pltpu.make_async_copy(tblref.at[gidx.at[jb + u]], rows[u],
                                      gsems[u]).wait()
                _scale_rows(rows[u], wv, jb + u)
                pltpu.async_copy(rows[u], acc.at[didx.at[jb + u]], ssems[u],
                                 add=True)
            return carry

        lax.fori_loop(0, RPG, rnd, 0)
        for u in range(NRING):
            pltpu.make_async_copy(rows[u], acc.at[didx.at[0]], ssems[u]).wait()
        return gcarry

    lax.fori_loop(0, NG, group, 0)


@functools.partial(
    pl.kernel,
    out_type=[
        jax.ShapeDtypeStruct((2 * N, DH), jnp.float32),
        jax.ShapeDtypeStruct((2 * N, DH), jnp.float32),
    ],
    mesh=_mesh,
    scratch_types=[
        pltpu.VMEM((GB, B), jnp.int32),
        pltpu.VMEM((GB, B), jnp.int32),
        pltpu.VMEM((GB, B), jnp.float32),
        [pltpu.VMEM((B, DH), jnp.float32) for _ in range(5)],
        pltpu.VMEM_SHARED((N, DH), jnp.float32),
        pltpu.VMEM_SHARED((N, DH), jnp.float32),
        [pltpu.SemaphoreType.DMA for _ in range(5)],
        [pltpu.SemaphoreType.DMA for _ in range(5)],
        pltpu.SemaphoreType.DMA,
    ],
    compiler_params=pltpu.CompilerParams(needs_layout_passes=False,
                                         use_tc_tiling_on_sc=False),
)
def _pair_prop(tbl, srcb, dstb, wb, p1_out, p2_out,
               gidx, didx, wv, rows, acc1, acc2,
               gsems, ssems, epsem):
    c = lax.axis_index("c")
    s = lax.axis_index("s")

    zv = jnp.zeros((16,), jnp.float32)

    def _zr(i, carry):
        for k in range(DH // 16):
            rows[0][i, pl.ds(k * 16, 16)] = zv
        return carry
    lax.fori_loop(0, B, _zr, 0)
    for p in range(EPR // B):
        pltpu.sync_copy(rows[0], acc1.at[pl.ds(s * EPR + p * B, B)])
        pltpu.sync_copy(rows[0], acc2.at[pl.ds(s * EPR + p * B, B)])
    rem = EPR - (EPR // B) * B
    if rem:
        pltpu.sync_copy(rows[0].at[pl.ds(0, rem)],
                        acc1.at[pl.ds(s * EPR + (EPR // B) * B, rem)])
        pltpu.sync_copy(rows[0].at[pl.ds(0, rem)],
                        acc2.at[pl.ds(s * EPR + (EPR // B) * B, rem)])

    @pl.when(s == NS - 1)
    def _():
        pltpu.sync_copy(rows[0].at[pl.ds(0, TAIL)],
                        acc1.at[pl.ds(NS * EPR, TAIL)])
        pltpu.sync_copy(rows[0].at[pl.ds(0, TAIL)],
                        acc2.at[pl.ds(NS * EPR, TAIL)])

    coff = c * N

    plsc.subcore_barrier()
    _prop_stage(tbl, srcb, dstb, wb, s, coff, gidx, didx, wv,
                rows, acc1, gsems, ssems)
    plsc.subcore_barrier()
    ep1 = pltpu.async_copy(acc1.at[pl.ds(s * EPR, EPR)],
                           p1_out.at[pl.ds(c * N + s * EPR, EPR)], epsem)

    @pl.when(s == NS - 1)
    def _():
        pltpu.sync_copy(acc1.at[pl.ds(NS * EPR, TAIL)],
                        p1_out.at[pl.ds(c * N + NS * EPR, TAIL)])

    _prop_stage(acc1, srcb, dstb, wb, s, c * 0, gidx, didx, wv,
                rows, acc2, gsems, ssems)
    ep1.wait()
    plsc.subcore_barrier()
    pltpu.sync_copy(acc2.at[pl.ds(s * EPR, EPR)],
                    p2_out.at[pl.ds(c * N + s * EPR, EPR)])

    @pl.when(s == NS - 1)
    def _():
        pltpu.sync_copy(acc2.at[pl.ds(NS * EPR, TAIL)],
                        p2_out.at[pl.ds(c * N + NS * EPR, TAIL)])


@functools.partial(
    pl.kernel,
    out_type=jax.ShapeDtypeStruct((NBALL, B), jnp.float32),
    mesh=_mesh,
    scratch_types=[
        pltpu.VMEM((NB, B), jnp.int32),
        pltpu.VMEM((NB, B), jnp.float32),
        pltpu.VMEM((B, 16), jnp.float32),
        pltpu.VMEM((B, 16), jnp.float32),
        pltpu.VMEM((DPT, 16), jnp.float32),
        pltpu.VMEM((DPT,), jnp.float32),
        pltpu.VMEM((NPD,), jnp.float32),
        pltpu.VMEM((WB_, B), jnp.int32),
        pltpu.VMEM((WB_, B), jnp.int32),
        pltpu.VMEM((WB_, B), jnp.float32),
        pltpu.VMEM((WB_, B), jnp.float32),
        pltpu.VMEM_SHARED((NPD, 16), jnp.float32),
        pltpu.VMEM_SHARED((NPD,), jnp.float32),
        pltpu.SemaphoreType.DMA,
        pltpu.SemaphoreType.DMA,
    ],
    compiler_params=pltpu.CompilerParams(needs_layout_passes=False,
                                         use_tc_tiling_on_sc=False),
)
def _edge_prep(srcb, dstb, ewb, w_out,
               sidx, ewv, rows0d, rows1d, deg16, disbuf, disfull,
               srcw, dstw, eww, wout, deg_acc, dis_sh, ssem0, ssem1):
    c = lax.axis_index("c")
    s = lax.axis_index("s")
    zv = jnp.zeros((16,), jnp.float32)

    def _z(i, carry):
        rows0d[i, pl.ds(0, 16)] = zv
        return carry
    lax.fori_loop(0, B, _z, 0)
    for k in range(DPT // B):
        pltpu.sync_copy(rows0d, deg_acc.at[pl.ds(s * DPT + k * B, B)])
    pltpu.sync_copy(srcb.at[pl.ds(s * NB, NB)], sidx)
    pltpu.sync_copy(ewb.at[pl.ds(s * NB, NB)], ewv)
    plsc.subcore_barrier()

    def _fill(rows, j):
        def fbody(g, carry):
            ev = ewv[j, pl.ds(g * 16, 16)]
            for u in range(16):
                rows[g * 16 + u, pl.ds(0, 16)] = ev[u] + zv
            return carry
        lax.fori_loop(0, B // 16, fbody, 0)

    def _dbody(p, carry):
        j0 = 2 * p
        j1 = j0 + 1
        _fill(rows0d, j0)
        pltpu.async_copy(rows0d, deg_acc.at[sidx.at[j0]], ssem0, add=True)

        @pl.when(p > 0)
        def _():
            pltpu.make_async_copy(rows1d, deg_acc.at[sidx.at[0]], ssem1).wait()

        _fill(rows1d, j1)
        pltpu.async_copy(rows1d, deg_acc.at[sidx.at[j1]], ssem1, add=True)
        pltpu.make_async_copy(rows0d, deg_acc.at[sidx.at[0]], ssem0).wait()
        return carry
    lax.fori_loop(0, NPAIR, _dbody, 0)
    pltpu.make_async_copy(rows1d, deg_acc.at[sidx.at[0]], ssem1).wait()
    plsc.subcore_barrier()

    pltpu.sync_copy(deg_acc.at[pl.ds(s * DPT, DPT)], deg16)
    czero = jnp.zeros((16,), jnp.int32)

    def _dis(i, carry):
        ridx = lax.iota(jnp.int32, 16) + i * 16
        d = plsc.load_gather(deg16, [ridx, czero])
        m = d > 0.0
        d1 = jnp.where(m, d, 1.0)
        ii = plsc.bitcast(d1, jnp.int32)
        ii = 0x5F3759DF - jnp.right_shift(ii, 1)
        y = plsc.bitcast(ii, jnp.float32)
        for _it in range(3):
            y = y * (1.5 - 0.5 * d1 * y * y)
        disbuf[pl.ds(i * 16, 16)] = jnp.where(m, y, 0.0)
        return carry
    lax.fori_loop(0, DPT // 16, _dis, 0)
    pltpu.sync_copy(disbuf, dis_sh.at[pl.ds(s * DPT, DPT)])
    plsc.subcore_barrier()
    pltpu.sync_copy(dis_sh, disfull)

    wbase = c * (E2 // B) + s * WB_
    pltpu.sync_copy(srcb.at[pl.ds(wbase, WB_)], srcw)
    pltpu.sync_copy(dstb.at[pl.ds(wbase, WB_)], dstw)
    pltpu.sync_copy(ewb.at[pl.ds(wbase, WB_)], eww)

    def _w(j, carry):
        for k in range(B // 16):
            sv = srcw[j, pl.ds(k * 16, 16)]
            dv = dstw[j, pl.ds(k * 16, 16)]
            ev = eww[j, pl.ds(k * 16, 16)]
            a = plsc.load_gather(disfull, [sv])
            bb = plsc.load_gather(disfull, [dv])
            wout[j, pl.ds(k * 16, 16)] = -(a * ev * bb)
        return carry
    lax.fori_loop(0, WB_, _w, 0)
    pltpu.sync_copy(wout, w_out.at[pl.ds(wbase, WB_)])


BN = 1000
NBLK = N // BN


def _dot(a, b):
    return lax.dot_general(a, b, (((1,), (0,)), ((), ())),
                           precision=lax.Precision.HIGHEST,
                           preferred_element_type=jnp.float32)


def _tc1_body(x, h, p1xl, p1xh, p2xl, p2xh, p1hl, p1hh, p2hl, p2hh,
              ax, a1, a2, bh, b1, b2, bias, z_ref, ht1_ref, hrs_ref):
    a1v = a1[...]
    a2v = a2[...]
    b1v = b1[...]
    b2v = b2[...]
    accx = (_dot(x[...], ax[...])
            + _dot(p1xl[...], a1v[:DH]) + _dot(p1xh[...], a1v[DH:])
            + _dot(p2xl[...], a2v[:DH]) + _dot(p2xh[...], a2v[DH:]))
    acch = (_dot(h[...], bh[...])
            + _dot(p1hl[...], b1v[:DH]) + _dot(p1hh[...], b1v[DH:])
            + _dot(p2hl[...], b2v[:DH]) + _dot(p2hh[...], b2v[DH:]))
    pre = accx + bias[...]
    z = jax.nn.sigmoid(pre[:, :D] + acch[:, :D])
    r = jax.nn.sigmoid(pre[:, D:2 * D] + acch[:, D:])
    z_ref[...] = z
    ht1_ref[...] = pre[:, 2 * D:]
    hr = h[...] * r
    hrs_ref[0] = hr[:, :DH]
    hrs_ref[1] = hr[:, DH:]


def _tc2_body(hrl, hrh, p1l, p1h_, p2l, p2h_, ht1, z, h,
              c0, c1, c2, bias, out_ref):
    c0v = c0[...]
    c1v = c1[...]
    c2v = c2[...]
    ht2 = (_dot(hrl[...], c0v[:DH]) + _dot(hrh[...], c0v[DH:])
           + _dot(p1l[...], c1v[:DH]) + _dot(p1h_[...], c1v[DH:])
           + _dot(p2l[...], c2v[:DH]) + _dot(p2h_[...], c2v[DH:])
           + bias[...])
    sacc = ht1[...] + ht2
    htil = jnp.tanh(sacc) + sacc
    zv = z[...]
    out_ref[...] = zv * h[...] + (1.0 - zv) * htil


def _rb(i):
    return (i, 0)


def _hb(i):
    return (i + NBLK, 0)


def _full(i):
    return (0, 0)


def _tc1(X, H, P1x, P2x, P1h, P2h, Ax, A1, A2, Bh, B1, B2, bias):
    return pl.pallas_call(
        _tc1_body,
        grid=(NBLK,),
        in_specs=[
            pl.BlockSpec((BN, D), _rb),
            pl.BlockSpec((BN, D), _rb),
            pl.BlockSpec((BN, DH), _rb), pl.BlockSpec((BN, DH), _hb),
            pl.BlockSpec((BN, DH), _rb), pl.BlockSpec((BN, DH), _hb),
            pl.BlockSpec((BN, DH), _rb), pl.BlockSpec((BN, DH), _hb),
            pl.BlockSpec((BN, DH), _rb), pl.BlockSpec((BN, DH), _hb),
            pl.BlockSpec((D, 3 * D), _full), pl.BlockSpec((D, 3 * D), _full),
            pl.BlockSpec((D, 3 * D), _full),
            pl.BlockSpec((D, 2 * D), _full), pl.BlockSpec((D, 2 * D), _full),
            pl.BlockSpec((D, 2 * D), _full),
            pl.BlockSpec((1, 3 * D), _full),
        ],
        out_specs=[
            pl.BlockSpec((BN, D), _rb),
            pl.BlockSpec((BN, D), _rb),
            pl.BlockSpec((2, BN, DH), lambda i: (0, i, 0)),
        ],
        out_shape=[
            jax.ShapeDtypeStruct((N, D), jnp.float32),
            jax.ShapeDtypeStruct((N, D), jnp.float32),
            jax.ShapeDtypeStruct((2, N, DH), jnp.float32),
        ],
    )(X, H, P1x, P1x, P2x, P2x, P1h, P1h, P2h, P2h,
      Ax, A1, A2, Bh, B1, B2, bias)


def _tc2(HRs, P1hr, P2hr, Ht1, Z, H, C0, C1, C2, bias):
    return pl.pallas_call(
        _tc2_body,
        grid=(NBLK,),
        in_specs=[
            pl.BlockSpec((BN, DH), _rb), pl.BlockSpec((BN, DH), _hb),
            pl.BlockSpec((BN, DH), _rb), pl.BlockSpec((BN, DH), _hb),
            pl.BlockSpec((BN, DH), _rb), pl.BlockSpec((BN, DH), _hb),
            pl.BlockSpec((BN, D), _rb),
            pl.BlockSpec((BN, D), _rb),
            pl.BlockSpec((BN, D), _rb),
            pl.BlockSpec((D, D), _full), pl.BlockSpec((D, D), _full),
            pl.BlockSpec((D, D), _full),
            pl.BlockSpec((1, D), _full),
        ],
        out_specs=pl.BlockSpec((BN, D), _rb),
        out_shape=jax.ShapeDtypeStruct((N, D), jnp.float32),
    )(HRs, HRs, P1hr, P1hr, P2hr, P2hr, Ht1, Z, H, C0, C1, C2, bias)


def kernel(X, edge_index, edge_weight, H,
           W_xz, b_xz, W_hz, b_hz, W_xr, b_xr, W_hr, b_hr,
           W_xh, b_xh, W_hh, b_hh):
    src = edge_index[0]
    dst = edge_index[1]
    srcb = src.reshape(NBALL, B)
    dstb = dst.reshape(NBALL, B)
    ewb = edge_weight.reshape(NBALL, B)

    wb = _edge_prep(srcb, dstb, ewb)

    Xs = jnp.concatenate([X[:, :DH], X[:, DH:]], axis=0)
    Hs = jnp.concatenate([H[:, :DH], H[:, DH:]], axis=0)
    P1x, P2x = _pair_prop(Xs, srcb, dstb, wb)
    P1h, P2h = _pair_prop(Hs, srcb, dstb, wb)

    Ax = jnp.concatenate([W_xz[0] - W_xz[2], W_xr[0] - W_xr[2],
                          W_xh[0] - W_xh[2]], axis=1)
    A1 = jnp.concatenate([W_xz[1], W_xr[1], W_xh[1]], axis=1)
    A2 = jnp.concatenate([2.0 * W_xz[2], 2.0 * W_xr[2], 2.0 * W_xh[2]], axis=1)
    Bh = jnp.concatenate([W_hz[0] - W_hz[2], W_hr[0] - W_hr[2]], axis=1)
    B1 = jnp.concatenate([W_hz[1], W_hr[1]], axis=1)
    B2 = jnp.concatenate([2.0 * W_hz[2], 2.0 * W_hr[2]], axis=1)
    bias = jnp.concatenate([b_xz + b_hz, b_xr + b_hr, b_xh])[None, :]

    Z, Ht1, HRs3 = _tc1(X, H, P1x, P2x, P1h, P2h, Ax, A1, A2, Bh, B1, B2, bias)
    HRs = HRs3.reshape(2 * N, DH)

    P1hr, P2hr = _pair_prop(HRs, srcb, dstb, wb)

    C0 = W_hh[0] - W_hh[2]
    C1 = W_hh[1]
    C2 = 2.0 * W_hh[2]
    return _tc2(HRs, P1hr, P2hr, Ht1, Z, H, C0, C1, C2, b_hh[None, :])


# speedup vs baseline: 1.1816x; 1.0282x over previous
import functools

import jax
import jax.numpy as jnp
from jax import lax
from jax.experimental import pallas as pl
from jax.experimental.pallas import tpu as pltpu
from jax.experimental.pallas import tpu_sc as plsc

N = 10000
E = 320000
D = 128
DH = 64
NC = 2
NS = 16
EPT = E // NS
B = 80
NB = EPT // B
NPAIR = NB // 2
EPR = 624
TAIL = N - NS * EPR
RZ = 208
NPD = 10240
DPT = NPD // NS
E2 = E // NC
WPT = E2 // NS
WB_ = WPT // B
NBALL = E // B

_mesh = plsc.VectorSubcoreMesh(core_axis_name="c", subcore_axis_name="s")


GB = 50
NG = NB // GB
PPG = GB // 2


def _scale_rows(rows, wv, j):
    def body(g, carry):
        wvec = wv[j, pl.ds(g * 16, 16)]
        for u in range(16):
            e = g * 16 + u
            we = wvec[u]
            for k in range(DH // 16):
                rows[e, pl.ds(k * 16, 16)] = rows[e, pl.ds(k * 16, 16)] * we
        return carry
    lax.fori_loop(0, B // 16, body, 0)


NRING = 5
RPG = GB // NRING


NHBM = 3


def _prop_stage(tblH, tblS, srcb, dstb, wb, s, coff, gidx, didx, wv,
                rows, acc, gsems, ssems):
    def group(g, gcarry):
        base = s * NB + g * GB
        pltpu.sync_copy(srcb.at[pl.ds(base, GB)], gidx)
        pltpu.sync_copy(dstb.at[pl.ds(base, GB)], didx)
        pltpu.sync_copy(wb.at[pl.ds(base, GB)], wv)

        def _adj(r, carry):
            for u in range(NHBM):
                j = r * NRING + u
                for k in range(B // 16):
                    gidx[j, pl.ds(k * 16, 16)] = (
                        gidx[j, pl.ds(k * 16, 16)] + coff)
            return carry
        lax.fori_loop(0, RPG, _adj, 0)

        def rnd(r, carry):
            jb = r * NRING
            for u in range(NRING):
                tblref = tblH if u < NHBM else tblS

                @pl.when(r > 0)
                def _():
                    pltpu.make_async_copy(rows[u], acc.at[didx.at[0]],
                                          ssems[u]).wait()
                pltpu.async_copy(tblref.at[gidx.at[jb + u]], rows[u], gsems[u])
            for u in range(NRING):
                tblref = tblH if u < NHBM else tblS
                pltpu.make_async_copy(tblref.at[gidx.at[jb + u]], rows[u],
                                      gsems[u]).wait()
                _scale_rows(rows[u], wv, jb + u)
                pltpu.async_copy(rows[u], acc.at[didx.at[jb + u]], ssems[u],
                                 add=True)
            return carry

        lax.fori_loop(0, RPG, rnd, 0)
        for u in range(NRING):
            pltpu.make_async_copy(rows[u], acc.at[didx.at[0]], ssems[u]).wait()
        return gcarry

    lax.fori_loop(0, NG, group, 0)


@functools.partial(
    pl.kernel,
    out_type=[
        jax.ShapeDtypeStruct((2 * N, DH), jnp.float32),
        jax.ShapeDtypeStruct((2 * N, DH), jnp.float32),
    ],
    mesh=_mesh,
    scratch_types=[
        pltpu.VMEM((GB, B), jnp.int32),
        pltpu.VMEM((GB, B), jnp.int32),
        pltpu.VMEM((GB, B), jnp.float32),
        [pltpu.VMEM((B, DH), jnp.float32) for _ in range(5)],
        pltpu.VMEM_SHARED((N, DH), jnp.float32),
        pltpu.VMEM_SHARED((N, DH), jnp.float32),
        [pltpu.SemaphoreType.DMA for _ in range(5)],
        [pltpu.SemaphoreType.DMA for _ in range(5)],
        pltpu.SemaphoreType.DMA,
    ],
    compiler_params=pltpu.CompilerParams(needs_layout_passes=False,
                                         use_tc_tiling_on_sc=False),
)
def _pair_prop(tbl, srcb, dstb, wb, p1_out, p2_out,
               gidx, didx, wv, rows, acc1, acc2,
               gsems, ssems, epsem):
    c = lax.axis_index("c")
    s = lax.axis_index("s")

    zv = jnp.zeros((16,), jnp.float32)

    def _zero_acc(accref):
        def _zr(i, carry):
            for k in range(DH // 16):
                rows[0][i, pl.ds(k * 16, 16)] = zv
            return carry
        lax.fori_loop(0, B, _zr, 0)
        for p in range(EPR // B):
            pltpu.sync_copy(rows[0], accref.at[pl.ds(s * EPR + p * B, B)])
        rem = EPR - (EPR // B) * B
        if rem:
            pltpu.sync_copy(rows[0].at[pl.ds(0, rem)],
                            accref.at[pl.ds(s * EPR + (EPR // B) * B, rem)])

        @pl.when(s == NS - 1)
        def _():
            pltpu.sync_copy(rows[0].at[pl.ds(0, TAIL)],
                            accref.at[pl.ds(NS * EPR, TAIL)])

    def _write_out(accref, outref):
        ep = pltpu.async_copy(accref.at[pl.ds(s * EPR, EPR)],
                              outref.at[pl.ds(c * N + s * EPR, EPR)], epsem)

        @pl.when(s == NS - 1)
        def _():
            pltpu.sync_copy(accref.at[pl.ds(NS * EPR, TAIL)],
                            outref.at[pl.ds(c * N + NS * EPR, TAIL)])

        ep.wait()

    coff = c * N

    _zero_acc(acc1)
    pltpu.sync_copy(tbl.at[pl.ds(c * N + s * EPR, EPR)],
                    acc2.at[pl.ds(s * EPR, EPR)])

    @pl.when(s == NS - 1)
    def _():
        pltpu.sync_copy(tbl.at[pl.ds(c * N + NS * EPR, TAIL)],
                        acc2.at[pl.ds(NS * EPR, TAIL)])

    plsc.subcore_barrier()
    _prop_stage(tbl, acc2, srcb, dstb, wb, s, coff, gidx, didx, wv,
                rows, acc1, gsems, ssems)
    plsc.subcore_barrier()
    _write_out(acc1, p1_out)
    _zero_acc(acc2)
    plsc.subcore_barrier()
    _prop_stage(p1_out, acc1, srcb, dstb, wb, s, coff, gidx, didx, wv,
                rows, acc2, gsems, ssems)
    plsc.subcore_barrier()
    _write_out(acc2, p2_out)


@functools.partial(
    pl.kernel,
    out_type=jax.ShapeDtypeStruct((NBALL, B), jnp.float32),
    mesh=_mesh,
    scratch_types=[
        pltpu.VMEM((NB, B), jnp.int32),
        pltpu.VMEM((NB, B), jnp.float32),
        pltpu.VMEM((B, 16), jnp.float32),
        pltpu.VMEM((B, 16), jnp.float32),
        pltpu.VMEM((DPT, 16), jnp.float32),
        pltpu.VMEM((DPT,), jnp.float32),
        pltpu.VMEM((NPD,), jnp.float32),
        pltpu.VMEM((WB_, B), jnp.int32),
        pltpu.VMEM((WB_, B), jnp.int32),
        pltpu.VMEM((WB_, B), jnp.float32),
        pltpu.VMEM((WB_, B), jnp.float32),
        pltpu.VMEM_SHARED((NPD, 16), jnp.float32),
        pltpu.VMEM_SHARED((NPD,), jnp.float32),
        pltpu.SemaphoreType.DMA,
        pltpu.SemaphoreType.DMA,
    ],
    compiler_params=pltpu.CompilerParams(needs_layout_passes=False,
                                         use_tc_tiling_on_sc=False),
)
def _edge_prep(srcb, dstb, ewb, w_out,
               sidx, ewv, rows0d, rows1d, deg16, disbuf, disfull,
               srcw, dstw, eww, wout, deg_acc, dis_sh, ssem0, ssem1):
    c = lax.axis_index("c")
    s = lax.axis_index("s")
    zv = jnp.zeros((16,), jnp.float32)

    def _z(i, carry):
        rows0d[i, pl.ds(0, 16)] = zv
        return carry
    lax.fori_loop(0, B, _z, 0)
    for k in range(DPT // B):
        pltpu.sync_copy(rows0d, deg_acc.at[pl.ds(s * DPT + k * B, B)])
    pltpu.sync_copy(srcb.at[pl.ds(s * NB, NB)], sidx)
    pltpu.sync_copy(ewb.at[pl.ds(s * NB, NB)], ewv)
    plsc.subcore_barrier()

    def _fill(rows, j):
        def fbody(g, carry):
            ev = ewv[j, pl.ds(g * 16, 16)]
            for u in range(16):
                rows[g * 16 + u, pl.ds(0, 16)] = ev[u] + zv
            return carry
        lax.fori_loop(0, B // 16, fbody, 0)

    def _dbody(p, carry):
        j0 = 2 * p
        j1 = j0 + 1
        _fill(rows0d, j0)
        pltpu.async_copy(rows0d, deg_acc.at[sidx.at[j0]], ssem0, add=True)

        @pl.when(p > 0)
        def _():
            pltpu.make_async_copy(rows1d, deg_acc.at[sidx.at[0]], ssem1).wait()

        _fill(rows1d, j1)
        pltpu.async_copy(rows1d, deg_acc.at[sidx.at[j1]], ssem1, add=True)
        pltpu.make_async_copy(rows0d, deg_acc.at[sidx.at[0]], ssem0).wait()
        return carry
    lax.fori_loop(0, NPAIR, _dbody, 0)
    pltpu.make_async_copy(rows1d, deg_acc.at[sidx.at[0]], ssem1).wait()
    plsc.subcore_barrier()

    pltpu.sync_copy(deg_acc.at[pl.ds(s * DPT, DPT)], deg16)
    czero = jnp.zeros((16,), jnp.int32)

    def _dis(i, carry):
        ridx = lax.iota(jnp.int32, 16) + i * 16
        d = plsc.load_gather(deg16, [ridx, czero])
        m = d > 0.0
        d1 = jnp.where(m, d, 1.0)
        ii = plsc.bitcast(d1, jnp.int32)
        ii = 0x5F3759DF - jnp.right_shift(ii, 1)
        y = plsc.bitcast(ii, jnp.float32)
        for _it in range(3):
            y = y * (1.5 - 0.5 * d1 * y * y)
        disbuf[pl.ds(i * 16, 16)] = jnp.where(m, y, 0.0)
        return carry
    lax.fori_loop(0, DPT // 16, _dis, 0)
    pltpu.sync_copy(disbuf, dis_sh.at[pl.ds(s * DPT, DPT)])
    plsc.subcore_barrier()
    pltpu.sync_copy(dis_sh, disfull)

    wbase = c * (E2 // B) + s * WB_
    pltpu.sync_copy(srcb.at[pl.ds(wbase, WB_)], srcw)
    pltpu.sync_copy(dstb.at[pl.ds(wbase, WB_)], dstw)
    pltpu.sync_copy(ewb.at[pl.ds(wbase, WB_)], eww)

    def _w(j, carry):
        for k in range(B // 16):
            sv = srcw[j, pl.ds(k * 16, 16)]
            dv = dstw[j, pl.ds(k * 16, 16)]
            ev = eww[j, pl.ds(k * 16, 16)]
            a = plsc.load_gather(disfull, [sv])
            bb = plsc.load_gather(disfull, [dv])
            wout[j, pl.ds(k * 16, 16)] = -(a * ev * bb)
        return carry
    lax.fori_loop(0, WB_, _w, 0)
    pltpu.sync_copy(wout, w_out.at[pl.ds(wbase, WB_)])


BN = 1000
NBLK = N // BN


def _dot(a, b):
    return lax.dot_general(a, b, (((1,), (0,)), ((), ())),
                           precision=lax.Precision.HIGHEST,
                           preferred_element_type=jnp.float32)


def _tc1_body(x, h, p1xl, p1xh, p2xl, p2xh, p1hl, p1hh, p2hl, p2hh,
              ax, a1, a2, bh, b1, b2, bias, z_ref, ht1_ref, hrs_ref):
    a1v = a1[...]
    a2v = a2[...]
    b1v = b1[...]
    b2v = b2[...]
    accx = (_dot(x[...], ax[...])
            + _dot(p1xl[...], a1v[:DH]) + _dot(p1xh[...], a1v[DH:])
            + _dot(p2xl[...], a2v[:DH]) + _dot(p2xh[...], a2v[DH:]))
    acch = (_dot(h[...], bh[...])
            + _dot(p1hl[...], b1v[:DH]) + _dot(p1hh[...], b1v[DH:])
            + _dot(p2hl[...], b2v[:DH]) + _dot(p2hh[...], b2v[DH:]))
    pre = accx + bias[...]
    z = jax.nn.sigmoid(pre[:, :D] + acch[:, :D])
    r = jax.nn.sigmoid(pre[:, D:2 * D] + acch[:, D:])
    z_ref[...] = z
    ht1_ref[...] = pre[:, 2 * D:]
    hr = h[...] * r
    hrs_ref[0] = hr[:, :DH]
    hrs_ref[1] = hr[:, DH:]


def _tc2_body(hrl, hrh, p1l, p1h_, p2l, p2h_, ht1, z, h,
              c0, c1, c2, bias, out_ref):
    c0v = c0[...]
    c1v = c1[...]
    c2v = c2[...]
    ht2 = (_dot(hrl[...], c0v[:DH]) + _dot(hrh[...], c0v[DH:])
           + _dot(p1l[...], c1v[:DH]) + _dot(p1h_[...], c1v[DH:])
           + _dot(p2l[...], c2v[:DH]) + _dot(p2h_[...], c2v[DH:])
           + bias[...])
    sacc = ht1[...] + ht2
    htil = jnp.tanh(sacc) + sacc
    zv = z[...]
    out_ref[...] = zv * h[...] + (1.0 - zv) * htil


def _rb(i):
    return (i, 0)


def _hb(i):
    return (i + NBLK, 0)


def _full(i):
    return (0, 0)


def _tc1(X, H, P1x, P2x, P1h, P2h, Ax, A1, A2, Bh, B1, B2, bias):
    return pl.pallas_call(
        _tc1_body,
        grid=(NBLK,),
        in_specs=[
            pl.BlockSpec((BN, D), _rb),
            pl.BlockSpec((BN, D), _rb),
            pl.BlockSpec((BN, DH), _rb), pl.BlockSpec((BN, DH), _hb),
            pl.BlockSpec((BN, DH), _rb), pl.BlockSpec((BN, DH), _hb),
            pl.BlockSpec((BN, DH), _rb), pl.BlockSpec((BN, DH), _hb),
            pl.BlockSpec((BN, DH), _rb), pl.BlockSpec((BN, DH), _hb),
            pl.BlockSpec((D, 3 * D), _full), pl.BlockSpec((D, 3 * D), _full),
            pl.BlockSpec((D, 3 * D), _full),
            pl.BlockSpec((D, 2 * D), _full), pl.BlockSpec((D, 2 * D), _full),
            pl.BlockSpec((D, 2 * D), _full),
            pl.BlockSpec((1, 3 * D), _full),
        ],
        out_specs=[
            pl.BlockSpec((BN, D), _rb),
            pl.BlockSpec((BN, D), _rb),
            pl.BlockSpec((2, BN, DH), lambda i: (0, i, 0)),
        ],
        out_shape=[
            jax.ShapeDtypeStruct((N, D), jnp.float32),
            jax.ShapeDtypeStruct((N, D), jnp.float32),
            jax.ShapeDtypeStruct((2, N, DH), jnp.float32),
        ],
    )(X, H, P1x, P1x, P2x, P2x, P1h, P1h, P2h, P2h,
      Ax, A1, A2, Bh, B1, B2, bias)


def _tc2(HRs, P1hr, P2hr, Ht1, Z, H, C0, C1, C2, bias):
    return pl.pallas_call(
        _tc2_body,
        grid=(NBLK,),
        in_specs=[
            pl.BlockSpec((BN, DH), _rb), pl.BlockSpec((BN, DH), _hb),
            pl.BlockSpec((BN, DH), _rb), pl.BlockSpec((BN, DH), _hb),
            pl.BlockSpec((BN, DH), _rb), pl.BlockSpec((BN, DH), _hb),
            pl.BlockSpec((BN, D), _rb),
            pl.BlockSpec((BN, D), _rb),
            pl.BlockSpec((BN, D), _rb),
            pl.BlockSpec((D, D), _full), pl.BlockSpec((D, D), _full),
            pl.BlockSpec((D, D), _full),
            pl.BlockSpec((1, D), _full),
        ],
        out_specs=pl.BlockSpec((BN, D), _rb),
        out_shape=jax.ShapeDtypeStruct((N, D), jnp.float32),
    )(HRs, HRs, P1hr, P1hr, P2hr, P2hr, Ht1, Z, H, C0, C1, C2, bias)


def kernel(X, edge_index, edge_weight, H,
           W_xz, b_xz, W_hz, b_hz, W_xr, b_xr, W_hr, b_hr,
           W_xh, b_xh, W_hh, b_hh):
    src = edge_index[0]
    dst = edge_index[1]
    srcb = src.reshape(NBALL, B)
    dstb = dst.reshape(NBALL, B)
    ewb = edge_weight.reshape(NBALL, B)

    wb = _edge_prep(srcb, dstb, ewb)

    Xs = jnp.concatenate([X[:, :DH], X[:, DH:]], axis=0)
    Hs = jnp.concatenate([H[:, :DH], H[:, DH:]], axis=0)
    P1x, P2x = _pair_prop(Xs, srcb, dstb, wb)
    P1h, P2h = _pair_prop(Hs, srcb, dstb, wb)

    Ax = jnp.concatenate([W_xz[0] - W_xz[2], W_xr[0] - W_xr[2],
                          W_xh[0] - W_xh[2]], axis=1)
    A1 = jnp.concatenate([W_xz[1], W_xr[1], W_xh[1]], axis=1)
    A2 = jnp.concatenate([2.0 * W_xz[2], 2.0 * W_xr[2], 2.0 * W_xh[2]], axis=1)
    Bh = jnp.concatenate([W_hz[0] - W_hz[2], W_hr[0] - W_hr[2]], axis=1)
    B1 = jnp.concatenate([W_hz[1], W_hr[1]], axis=1)
    B2 = jnp.concatenate([2.0 * W_hz[2], 2.0 * W_hr[2]], axis=1)
    bias = jnp.concatenate([b_xz + b_hz, b_xr + b_hr, b_xh])[None, :]

    Z, Ht1, HRs3 = _tc1(X, H, P1x, P2x, P1h, P2h, Ax, A1, A2, Bh, B1, B2, bias)
    HRs = HRs3.reshape(2 * N, DH)

    P1hr, P2hr = _pair_prop(HRs, srcb, dstb, wb)

    C0 = W_hh[0] - W_hh[2]
    C1 = W_hh[1]
    C2 = 2.0 * W_hh[2]
    return _tc2(HRs, P1hr, P2hr, Ht1, Z, H, C0, C1, C2, b_hh[None, :])


# speedup vs baseline: 1.2938x; 1.0950x over previous
import functools

import jax
import jax.numpy as jnp
from jax import lax
from jax.experimental import pallas as pl
from jax.experimental.pallas import tpu as pltpu
from jax.experimental.pallas import tpu_sc as plsc

N = 10000
E = 320000
D = 128
DH = 64
NC = 2
NS = 16
EPT = E // NS
B = 80
NB = EPT // B
NPAIR = NB // 2
EPR = 624
TAIL = N - NS * EPR
RZ = 208
NPD = 10240
DPT = NPD // NS
E2 = E // NC
WPT = E2 // NS
WB_ = WPT // B
NBALL = E // B

_mesh = plsc.VectorSubcoreMesh(core_axis_name="c", subcore_axis_name="s")


GB = 50
NG = NB // GB
PPG = GB // 2


def _scale_rows(rows, wv, j):
    def body(g, carry):
        wvec = wv[j, pl.ds(g * 16, 16)]
        for u in range(16):
            e = g * 16 + u
            we = wvec[u]
            for k in range(DH // 16):
                rows[e, pl.ds(k * 16, 16)] = rows[e, pl.ds(k * 16, 16)] * we
        return carry
    lax.fori_loop(0, B // 16, body, 0)


NRING = 5
RPG = GB // NRING


NHBM = 3


def _prop_stage(tblH, tblS, srcb, dstb, wb, s, coff, gidx, didx, wv,
                rows, acc, gsems, ssems):
    def group(g, gcarry):
        base = s * NB + g * GB
        pltpu.sync_copy(srcb.at[pl.ds(base, GB)], gidx)
        pltpu.sync_copy(dstb.at[pl.ds(base, GB)], didx)
        pltpu.sync_copy(wb.at[pl.ds(base, GB)], wv)

        def _adj(r, carry):
            for u in range(NHBM):
                j = r * NRING + u
                for k in range(B // 16):
                    gidx[j, pl.ds(k * 16, 16)] = (
                        gidx[j, pl.ds(k * 16, 16)] + coff)
            return carry
        lax.fori_loop(0, RPG, _adj, 0)

        def rnd(r, carry):
            jb = r * NRING
            for u in range(NRING):
                tblref = tblH if u < NHBM else tblS

                @pl.when(r > 0)
                def _():
                    pltpu.make_async_copy(rows[u], acc.at[didx.at[0]],
                                          ssems[u]).wait()
                pltpu.async_copy(tblref.at[gidx.at[jb + u]], rows[u], gsems[u])
            for u in range(NRING):
                tblref = tblH if u < NHBM else tblS
                pltpu.make_async_copy(tblref.at[gidx.at[jb + u]], rows[u],
                                      gsems[u]).wait()
                _scale_rows(rows[u], wv, jb + u)
                pltpu.async_copy(rows[u], acc.at[didx.at[jb + u]], ssems[u],
                                 add=True)
            return carry

        lax.fori_loop(0, RPG, rnd, 0)
        for u in range(NRING):
            pltpu.make_async_copy(rows[u], acc.at[didx.at[0]], ssems[u]).wait()
        return gcarry

    lax.fori_loop(0, NG, group, 0)


def _make_prop_kernel(ntbl):
  out1 = [jax.ShapeDtypeStruct((2 * N, DH), jnp.float32)] * (2 * ntbl)
  return functools.partial(
    pl.kernel,
    out_type=out1,
    mesh=_mesh,
    scratch_types=[
        pltpu.VMEM((GB, B), jnp.int32),
        pltpu.VMEM((GB, B), jnp.int32),
        pltpu.VMEM((GB, B), jnp.float32),
        [pltpu.VMEM((B, DH), jnp.float32) for _ in range(5)],
        pltpu.VMEM_SHARED((N, DH), jnp.float32),
        pltpu.VMEM_SHARED((N, DH), jnp.float32),
        [pltpu.SemaphoreType.DMA for _ in range(5)],
        [pltpu.SemaphoreType.DMA for _ in range(5)],
        pltpu.SemaphoreType.DMA,
    ],
    compiler_params=pltpu.CompilerParams(needs_layout_passes=False,
                                         use_tc_tiling_on_sc=False),
  )


def _prop_body(tbls, srcb, dstb, wb, outs,
               gidx, didx, wv, rows, acc1, acc2, gsems, ssems, epsem):
    c = lax.axis_index("c")
    s = lax.axis_index("s")

    zv = jnp.zeros((16,), jnp.float32)

    def _zero_acc(accref):
        def _zr(i, carry):
            for k in range(DH // 16):
                rows[0][i, pl.ds(k * 16, 16)] = zv
            return carry
        lax.fori_loop(0, B, _zr, 0)
        for p in range(EPR // B):
            pltpu.sync_copy(rows[0], accref.at[pl.ds(s * EPR + p * B, B)])
        rem = EPR - (EPR // B) * B
        if rem:
            pltpu.sync_copy(rows[0].at[pl.ds(0, rem)],
                            accref.at[pl.ds(s * EPR + (EPR // B) * B, rem)])

        @pl.when(s == NS - 1)
        def _():
            pltpu.sync_copy(rows[0].at[pl.ds(0, TAIL)],
                            accref.at[pl.ds(NS * EPR, TAIL)])

    def _write_out(accref, outref):
        ep = pltpu.async_copy(accref.at[pl.ds(s * EPR, EPR)],
                              outref.at[pl.ds(c * N + s * EPR, EPR)], epsem)

        @pl.when(s == NS - 1)
        def _():
            pltpu.sync_copy(accref.at[pl.ds(NS * EPR, TAIL)],
                            outref.at[pl.ds(c * N + NS * EPR, TAIL)])

        ep.wait()

    coff = c * N

    for t, tbl in enumerate(tbls):
        p1_out = outs[2 * t]
        p2_out = outs[2 * t + 1]
        _zero_acc(acc1)
        pltpu.sync_copy(tbl.at[pl.ds(c * N + s * EPR, EPR)],
                        acc2.at[pl.ds(s * EPR, EPR)])

        @pl.when(s == NS - 1)
        def _():
            pltpu.sync_copy(tbl.at[pl.ds(c * N + NS * EPR, TAIL)],
                            acc2.at[pl.ds(NS * EPR, TAIL)])

        plsc.subcore_barrier()
        _prop_stage(tbl, acc2, srcb, dstb, wb, s, coff, gidx, didx, wv,
                    rows, acc1, gsems, ssems)
        plsc.subcore_barrier()
        _write_out(acc1, p1_out)
        _zero_acc(acc2)
        plsc.subcore_barrier()
        _prop_stage(p1_out, acc1, srcb, dstb, wb, s, coff, gidx, didx, wv,
                    rows, acc2, gsems, ssems)
        plsc.subcore_barrier()
        _write_out(acc2, p2_out)
        plsc.subcore_barrier()


@_make_prop_kernel(2)
def _quad_prop(tblA, tblB, srcb, dstb, wb, o1, o2, o3, o4,
               gidx, didx, wv, rows, acc1, acc2, gsems, ssems, epsem):
    _prop_body((tblA, tblB), srcb, dstb, wb, (o1, o2, o3, o4),
               gidx, didx, wv, rows, acc1, acc2, gsems, ssems, epsem)


@_make_prop_kernel(1)
def _pair_prop(tbl, srcb, dstb, wb, o1, o2,
               gidx, didx, wv, rows, acc1, acc2, gsems, ssems, epsem):
    _prop_body((tbl,), srcb, dstb, wb, (o1, o2),
               gidx, didx, wv, rows, acc1, acc2, gsems, ssems, epsem)


@functools.partial(
    pl.kernel,
    out_type=jax.ShapeDtypeStruct((NBALL, B), jnp.float32),
    mesh=_mesh,
    scratch_types=[
        pltpu.VMEM((NB, B), jnp.int32),
        pltpu.VMEM((NB, B), jnp.float32),
        pltpu.VMEM((B, 16), jnp.float32),
        pltpu.VMEM((B, 16), jnp.float32),
        pltpu.VMEM((DPT, 16), jnp.float32),
        pltpu.VMEM((DPT,), jnp.float32),
        pltpu.VMEM((NPD,), jnp.float32),
        pltpu.VMEM((WB_, B), jnp.int32),
        pltpu.VMEM((WB_, B), jnp.int32),
        pltpu.VMEM((WB_, B), jnp.float32),
        pltpu.VMEM((WB_, B), jnp.float32),
        pltpu.VMEM_SHARED((NPD, 16), jnp.float32),
        pltpu.VMEM_SHARED((NPD,), jnp.float32),
        pltpu.SemaphoreType.DMA,
        pltpu.SemaphoreType.DMA,
    ],
    compiler_params=pltpu.CompilerParams(needs_layout_passes=False,
                                         use_tc_tiling_on_sc=False),
)
def _edge_prep(srcb, dstb, ewb, w_out,
               sidx, ewv, rows0d, rows1d, deg16, disbuf, disfull,
               srcw, dstw, eww, wout, deg_acc, dis_sh, ssem0, ssem1):
    c = lax.axis_index("c")
    s = lax.axis_index("s")
    zv = jnp.zeros((16,), jnp.float32)

    def _z(i, carry):
        rows0d[i, pl.ds(0, 16)] = zv
        return carry
    lax.fori_loop(0, B, _z, 0)
    for k in range(DPT // B):
        pltpu.sync_copy(rows0d, deg_acc.at[pl.ds(s * DPT + k * B, B)])
    pltpu.sync_copy(srcb.at[pl.ds(s * NB, NB)], sidx)
    pltpu.sync_copy(ewb.at[pl.ds(s * NB, NB)], ewv)
    plsc.subcore_barrier()

    def _fill(rows, j):
        def fbody(g, carry):
            ev = ewv[j, pl.ds(g * 16, 16)]
            for u in range(16):
                rows[g * 16 + u, pl.ds(0, 16)] = ev[u] + zv
            return carry
        lax.fori_loop(0, B // 16, fbody, 0)

    def _dbody(p, carry):
        j0 = 2 * p
        j1 = j0 + 1
        _fill(rows0d, j0)
        pltpu.async_copy(rows0d, deg_acc.at[sidx.at[j0]], ssem0, add=True)

        @pl.when(p > 0)
        def _():
            pltpu.make_async_copy(rows1d, deg_acc.at[sidx.at[0]], ssem1).wait()

        _fill(rows1d, j1)
        pltpu.async_copy(rows1d, deg_acc.at[sidx.at[j1]], ssem1, add=True)
        pltpu.make_async_copy(rows0d, deg_acc.at[sidx.at[0]], ssem0).wait()
        return carry
    lax.fori_loop(0, NPAIR, _dbody, 0)
    pltpu.make_async_copy(rows1d, deg_acc.at[sidx.at[0]], ssem1).wait()
    plsc.subcore_barrier()

    pltpu.sync_copy(deg_acc.at[pl.ds(s * DPT, DPT)], deg16)
    czero = jnp.zeros((16,), jnp.int32)

    def _dis(i, carry):
        ridx = lax.iota(jnp.int32, 16) + i * 16
        d = plsc.load_gather(deg16, [ridx, czero])
        m = d > 0.0
        d1 = jnp.where(m, d, 1.0)
        ii = plsc.bitcast(d1, jnp.int32)
        ii = 0x5F3759DF - jnp.right_shift(ii, 1)
        y = plsc.bitcast(ii, jnp.float32)
        for _it in range(3):
            y = y * (1.5 - 0.5 * d1 * y * y)
        disbuf[pl.ds(i * 16, 16)] = jnp.where(m, y, 0.0)
        return carry
    lax.fori_loop(0, DPT // 16, _dis, 0)
    pltpu.sync_copy(disbuf, dis_sh.at[pl.ds(s * DPT, DPT)])
    plsc.subcore_barrier()
    pltpu.sync_copy(dis_sh, disfull)

    wbase = c * (E2 // B) + s * WB_
    pltpu.sync_copy(srcb.at[pl.ds(wbase, WB_)], srcw)
    pltpu.sync_copy(dstb.at[pl.ds(wbase, WB_)], dstw)
    pltpu.sync_copy(ewb.at[pl.ds(wbase, WB_)], eww)

    def _w(j, carry):
        for k in range(B // 16):
            sv = srcw[j, pl.ds(k * 16, 16)]
            dv = dstw[j, pl.ds(k * 16, 16)]
            ev = eww[j, pl.ds(k * 16, 16)]
            a = plsc.load_gather(disfull, [sv])
            bb = plsc.load_gather(disfull, [dv])
            wout[j, pl.ds(k * 16, 16)] = -(a * ev * bb)
        return carry
    lax.fori_loop(0, WB_, _w, 0)
    pltpu.sync_copy(wout, w_out.at[pl.ds(wbase, WB_)])


BN = 1000
NBLK = N // BN


def _dot(a, b):
    return lax.dot_general(a, b, (((1,), (0,)), ((), ())),
                           preferred_element_type=jnp.float32)


def _tc1_body(x, h, p1xl, p1xh, p2xl, p2xh, p1hl, p1hh, p2hl, p2hh,
              ax, a1, a2, bh, b1, b2, bias, z_ref, ht1_ref, hrs_ref):
    a1v = a1[...]
    a2v = a2[...]
    b1v = b1[...]
    b2v = b2[...]
    accx = (_dot(x[...], ax[...])
            + _dot(p1xl[...], a1v[:DH]) + _dot(p1xh[...], a1v[DH:])
            + _dot(p2xl[...], a2v[:DH]) + _dot(p2xh[...], a2v[DH:]))
    acch = (_dot(h[...], bh[...])
            + _dot(p1hl[...], b1v[:DH]) + _dot(p1hh[...], b1v[DH:])
            + _dot(p2hl[...], b2v[:DH]) + _dot(p2hh[...], b2v[DH:]))
    pre = accx + bias[...]
    z = jax.nn.sigmoid(pre[:, :D] + acch[:, :D])
    r = jax.nn.sigmoid(pre[:, D:2 * D] + acch[:, D:])
    z_ref[...] = z
    ht1_ref[...] = pre[:, 2 * D:]
    hr = h[...] * r
    hrs_ref[0] = hr[:, :DH]
    hrs_ref[1] = hr[:, DH:]


def _tc2_body(hrl, hrh, p1l, p1h_, p2l, p2h_, ht1, z, h,
              c0, c1, c2, bias, out_ref):
    c0v = c0[...]
    c1v = c1[...]
    c2v = c2[...]
    ht2 = (_dot(hrl[...], c0v[:DH]) + _dot(hrh[...], c0v[DH:])
           + _dot(p1l[...], c1v[:DH]) + _dot(p1h_[...], c1v[DH:])
           + _dot(p2l[...], c2v[:DH]) + _dot(p2h_[...], c2v[DH:])
           + bias[...])
    sacc = ht1[...] + ht2
    htil = jnp.tanh(sacc) + sacc
    zv = z[...]
    out_ref[...] = zv * h[...] + (1.0 - zv) * htil


def _rb(i):
    return (i, 0)


def _hb(i):
    return (i + NBLK, 0)


def _full(i):
    return (0, 0)


def _tc1(X, H, P1x, P2x, P1h, P2h, Ax, A1, A2, Bh, B1, B2, bias):
    return pl.pallas_call(
        _tc1_body,
        grid=(NBLK,),
        in_specs=[
            pl.BlockSpec((BN, D), _rb),
            pl.BlockSpec((BN, D), _rb),
            pl.BlockSpec((BN, DH), _rb), pl.BlockSpec((BN, DH), _hb),
            pl.BlockSpec((BN, DH), _rb), pl.BlockSpec((BN, DH), _hb),
            pl.BlockSpec((BN, DH), _rb), pl.BlockSpec((BN, DH), _hb),
            pl.BlockSpec((BN, DH), _rb), pl.BlockSpec((BN, DH), _hb),
            pl.BlockSpec((D, 3 * D), _full), pl.BlockSpec((D, 3 * D), _full),
            pl.BlockSpec((D, 3 * D), _full),
            pl.BlockSpec((D, 2 * D), _full), pl.BlockSpec((D, 2 * D), _full),
            pl.BlockSpec((D, 2 * D), _full),
            pl.BlockSpec((1, 3 * D), _full),
        ],
        out_specs=[
            pl.BlockSpec((BN, D), _rb),
            pl.BlockSpec((BN, D), _rb),
            pl.BlockSpec((2, BN, DH), lambda i: (0, i, 0)),
        ],
        out_shape=[
            jax.ShapeDtypeStruct((N, D), jnp.float32),
            jax.ShapeDtypeStruct((N, D), jnp.float32),
            jax.ShapeDtypeStruct((2, N, DH), jnp.float32),
        ],
    )(X, H, P1x, P1x, P2x, P2x, P1h, P1h, P2h, P2h,
      Ax, A1, A2, Bh, B1, B2, bias)


def _tc2(HRs, P1hr, P2hr, Ht1, Z, H, C0, C1, C2, bias):
    return pl.pallas_call(
        _tc2_body,
        grid=(NBLK,),
        in_specs=[
            pl.BlockSpec((BN, DH), _rb), pl.BlockSpec((BN, DH), _hb),
            pl.BlockSpec((BN, DH), _rb), pl.BlockSpec((BN, DH), _hb),
            pl.BlockSpec((BN, DH), _rb), pl.BlockSpec((BN, DH), _hb),
            pl.BlockSpec((BN, D), _rb),
            pl.BlockSpec((BN, D), _rb),
            pl.BlockSpec((BN, D), _rb),
            pl.BlockSpec((D, D), _full), pl.BlockSpec((D, D), _full),
            pl.BlockSpec((D, D), _full),
            pl.BlockSpec((1, D), _full),
        ],
        out_specs=pl.BlockSpec((BN, D), _rb),
        out_shape=jax.ShapeDtypeStruct((N, D), jnp.float32),
    )(HRs, HRs, P1hr, P1hr, P2hr, P2hr, Ht1, Z, H, C0, C1, C2, bias)


def kernel(X, edge_index, edge_weight, H,
           W_xz, b_xz, W_hz, b_hz, W_xr, b_xr, W_hr, b_hr,
           W_xh, b_xh, W_hh, b_hh):
    src = edge_index[0]
    dst = edge_index[1]
    srcb = src.reshape(NBALL, B)
    dstb = dst.reshape(NBALL, B)
    ewb = edge_weight.reshape(NBALL, B)

    wb = _edge_prep(srcb, dstb, ewb)

    Xs = jnp.concatenate([X[:, :DH], X[:, DH:]], axis=0)
    Hs = jnp.concatenate([H[:, :DH], H[:, DH:]], axis=0)
    P1x, P2x, P1h, P2h = _quad_prop(Xs, Hs, srcb, dstb, wb)

    Ax = jnp.concatenate([W_xz[0] - W_xz[2], W_xr[0] - W_xr[2],
                          W_xh[0] - W_xh[2]], axis=1)
    A1 = jnp.concatenate([W_xz[1], W_xr[1], W_xh[1]], axis=1)
    A2 = jnp.concatenate([2.0 * W_xz[2], 2.0 * W_xr[2], 2.0 * W_xh[2]], axis=1)
    Bh = jnp.concatenate([W_hz[0] - W_hz[2], W_hr[0] - W_hr[2]], axis=1)
    B1 = jnp.concatenate([W_hz[1], W_hr[1]], axis=1)
    B2 = jnp.concatenate([2.0 * W_hz[2], 2.0 * W_hr[2]], axis=1)
    bias = jnp.concatenate([b_xz + b_hz, b_xr + b_hr, b_xh])[None, :]

    Z, Ht1, HRs3 = _tc1(X, H, P1x, P2x, P1h, P2h, Ax, A1, A2, Bh, B1, B2, bias)
    HRs = HRs3.reshape(2 * N, DH)

    P1hr, P2hr = _pair_prop(HRs, srcb, dstb, wb)

    C0 = W_hh[0] - W_hh[2]
    C1 = W_hh[1]
    C2 = 2.0 * W_hh[2]
    return _tc2(HRs, P1hr, P2hr, Ht1, Z, H, C0, C1, C2, b_hh[None, :])


# speedup vs baseline: 1.4984x; 1.1581x over previous
import functools

import jax
import jax.numpy as jnp
from jax import lax
from jax.experimental import pallas as pl
from jax.experimental.pallas import tpu as pltpu
from jax.experimental.pallas import tpu_sc as plsc

N = 10000
E = 320000
D = 128
DH = 64
NC = 2
NS = 16
EPT = E // NS
B = 80
NB = EPT // B
NPAIR = NB // 2
EPR = 624
TAIL = N - NS * EPR
RZ = 208
NPD = 10240
DPT = NPD // NS
E2 = E // NC
WPT = E2 // NS
WB_ = WPT // B
NBALL = E // B

_mesh = plsc.VectorSubcoreMesh(core_axis_name="c", subcore_axis_name="s")


GB = 50
NG = NB // GB
PPG = GB // 2


def _scale_rows(rows, wv, j):
    @functools.partial(plsc.parallel_loop, 0, B // 16)
    def body(g):
        wvec = wv[j, pl.ds(g * 16, 16)]
        for u in range(16):
            e = g * 16 + u
            we = wvec[u]
            for k in range(DH // 16):
                rows[e, pl.ds(k * 16, 16)] = rows[e, pl.ds(k * 16, 16)] * we


NRING = 5
RPG = GB // NRING


NHBM = 3


def _prop_stage(tblH, tblS, srcb, dstb, wb, s, coff, gidx, didx, wv,
                rows, acc, gsems, ssems):
    def group(g, gcarry):
        base = s * NB + g * GB
        pltpu.sync_copy(srcb.at[pl.ds(base, GB)], gidx)
        pltpu.sync_copy(dstb.at[pl.ds(base, GB)], didx)
        pltpu.sync_copy(wb.at[pl.ds(base, GB)], wv)

        def _adj(r, carry):
            for u in range(NHBM):
                j = r * NRING + u
                for k in range(B // 16):
                    gidx[j, pl.ds(k * 16, 16)] = (
                        gidx[j, pl.ds(k * 16, 16)] + coff)
            return carry
        lax.fori_loop(0, RPG, _adj, 0)

        def rnd(r, carry):
            jb = r * NRING
            for u in range(NRING):
                tblref = tblH if u < NHBM else tblS

                @pl.when(r > 0)
                def _():
                    pltpu.make_async_copy(rows[u], acc.at[didx.at[0]],
                                          ssems[u]).wait()
                pltpu.async_copy(tblref.at[gidx.at[jb + u]], rows[u], gsems[u])
            for u in range(NRING):
                tblref = tblH if u < NHBM else tblS
                pltpu.make_async_copy(tblref.at[gidx.at[jb + u]], rows[u],
                                      gsems[u]).wait()
                _scale_rows(rows[u], wv, jb + u)
                pltpu.async_copy(rows[u], acc.at[didx.at[jb + u]], ssems[u],
                                 add=True)
            return carry

        lax.fori_loop(0, RPG, rnd, 0)
        for u in range(NRING):
            pltpu.make_async_copy(rows[u], acc.at[didx.at[0]], ssems[u]).wait()
        return gcarry

    lax.fori_loop(0, NG, group, 0)


def _make_prop_kernel(ntbl):
  out1 = [jax.ShapeDtypeStruct((2 * N, DH), jnp.float32)] * (2 * ntbl)
  return functools.partial(
    pl.kernel,
    out_type=out1,
    mesh=_mesh,
    scratch_types=[
        pltpu.VMEM((GB, B), jnp.int32),
        pltpu.VMEM((GB, B), jnp.int32),
        pltpu.VMEM((GB, B), jnp.float32),
        [pltpu.VMEM((B, DH), jnp.float32) for _ in range(5)],
        pltpu.VMEM_SHARED((N, DH), jnp.float32),
        pltpu.VMEM_SHARED((N, DH), jnp.float32),
        [pltpu.SemaphoreType.DMA for _ in range(5)],
        [pltpu.SemaphoreType.DMA for _ in range(5)],
        pltpu.SemaphoreType.DMA,
    ],
    compiler_params=pltpu.CompilerParams(needs_layout_passes=False,
                                         use_tc_tiling_on_sc=False),
  )


def _prop_body(tbls, srcb, dstb, wb, outs,
               gidx, didx, wv, rows, acc1, acc2, gsems, ssems, epsem):
    c = lax.axis_index("c")
    s = lax.axis_index("s")

    zv = jnp.zeros((16,), jnp.float32)

    def _zero_acc(accref):
        def _zr(i, carry):
            for k in range(DH // 16):
                rows[0][i, pl.ds(k * 16, 16)] = zv
            return carry
        lax.fori_loop(0, B, _zr, 0)
        for p in range(EPR // B):
            pltpu.sync_copy(rows[0], accref.at[pl.ds(s * EPR + p * B, B)])
        rem = EPR - (EPR // B) * B
        if rem:
            pltpu.sync_copy(rows[0].at[pl.ds(0, rem)],
                            accref.at[pl.ds(s * EPR + (EPR // B) * B, rem)])

        @pl.when(s == NS - 1)
        def _():
            pltpu.sync_copy(rows[0].at[pl.ds(0, TAIL)],
                            accref.at[pl.ds(NS * EPR, TAIL)])

    def _write_out(accref, outref):
        ep = pltpu.async_copy(accref.at[pl.ds(s * EPR, EPR)],
                              outref.at[pl.ds(c * N + s * EPR, EPR)], epsem)

        @pl.when(s == NS - 1)
        def _():
            pltpu.sync_copy(accref.at[pl.ds(NS * EPR, TAIL)],
                            outref.at[pl.ds(c * N + NS * EPR, TAIL)])

        ep.wait()

    coff = c * N

    for t, tbl in enumerate(tbls):
        p1_out = outs[2 * t]
        p2_out = outs[2 * t + 1]
        _zero_acc(acc1)
        pltpu.sync_copy(tbl.at[pl.ds(c * N + s * EPR, EPR)],
                        acc2.at[pl.ds(s * EPR, EPR)])

        @pl.when(s == NS - 1)
        def _():
            pltpu.sync_copy(tbl.at[pl.ds(c * N + NS * EPR, TAIL)],
                            acc2.at[pl.ds(NS * EPR, TAIL)])

        plsc.subcore_barrier()
        _prop_stage(tbl, acc2, srcb, dstb, wb, s, coff, gidx, didx, wv,
                    rows, acc1, gsems, ssems)
        plsc.subcore_barrier()
        _write_out(acc1, p1_out)
        _zero_acc(acc2)
        plsc.subcore_barrier()
        _prop_stage(p1_out, acc1, srcb, dstb, wb, s, coff, gidx, didx, wv,
                    rows, acc2, gsems, ssems)
        plsc.subcore_barrier()
        _write_out(acc2, p2_out)
        plsc.subcore_barrier()


@_make_prop_kernel(2)
def _quad_prop(tblA, tblB, srcb, dstb, wb, o1, o2, o3, o4,
               gidx, didx, wv, rows, acc1, acc2, gsems, ssems, epsem):
    _prop_body((tblA, tblB), srcb, dstb, wb, (o1, o2, o3, o4),
               gidx, didx, wv, rows, acc1, acc2, gsems, ssems, epsem)


@_make_prop_kernel(1)
def _pair_prop(tbl, srcb, dstb, wb, o1, o2,
               gidx, didx, wv, rows, acc1, acc2, gsems, ssems, epsem):
    _prop_body((tbl,), srcb, dstb, wb, (o1, o2),
               gidx, didx, wv, rows, acc1, acc2, gsems, ssems, epsem)


@functools.partial(
    pl.kernel,
    out_type=jax.ShapeDtypeStruct((NBALL, B), jnp.float32),
    mesh=_mesh,
    scratch_types=[
        pltpu.VMEM((NB, B), jnp.int32),
        pltpu.VMEM((NB, B), jnp.float32),
        pltpu.VMEM((B, 16), jnp.float32),
        pltpu.VMEM((B, 16), jnp.float32),
        pltpu.VMEM((DPT, 16), jnp.float32),
        pltpu.VMEM((DPT,), jnp.float32),
        pltpu.VMEM((NPD,), jnp.float32),
        pltpu.VMEM((WB_, B), jnp.int32),
        pltpu.VMEM((WB_, B), jnp.int32),
        pltpu.VMEM((WB_, B), jnp.float32),
        pltpu.VMEM((WB_, B), jnp.float32),
        pltpu.VMEM_SHARED((NPD, 16), jnp.float32),
        pltpu.VMEM_SHARED((NPD,), jnp.float32),
        pltpu.SemaphoreType.DMA,
        pltpu.SemaphoreType.DMA,
    ],
    compiler_params=pltpu.CompilerParams(needs_layout_passes=False,
                                         use_tc_tiling_on_sc=False),
)
def _edge_prep(srcb, dstb, ewb, w_out,
               sidx, ewv, rows0d, rows1d, deg16, disbuf, disfull,
               srcw, dstw, eww, wout, deg_acc, dis_sh, ssem0, ssem1):
    c = lax.axis_index("c")
    s = lax.axis_index("s")
    zv = jnp.zeros((16,), jnp.float32)

    def _z(i, carry):
        rows0d[i, pl.ds(0, 16)] = zv
        return carry
    lax.fori_loop(0, B, _z, 0)
    for k in range(DPT // B):
        pltpu.sync_copy(rows0d, deg_acc.at[pl.ds(s * DPT + k * B, B)])
    pltpu.sync_copy(srcb.at[pl.ds(s * NB, NB)], sidx)
    pltpu.sync_copy(ewb.at[pl.ds(s * NB, NB)], ewv)
    plsc.subcore_barrier()

    def _fill(rows, j):
        def fbody(g, carry):
            ev = ewv[j, pl.ds(g * 16, 16)]
            for u in range(16):
                rows[g * 16 + u, pl.ds(0, 16)] = ev[u] + zv
            return carry
        lax.fori_loop(0, B // 16, fbody, 0)

    def _dbody(p, carry):
        j0 = 2 * p
        j1 = j0 + 1
        _fill(rows0d, j0)
        pltpu.async_copy(rows0d, deg_acc.at[sidx.at[j0]], ssem0, add=True)

        @pl.when(p > 0)
        def _():
            pltpu.make_async_copy(rows1d, deg_acc.at[sidx.at[0]], ssem1).wait()

        _fill(rows1d, j1)
        pltpu.async_copy(rows1d, deg_acc.at[sidx.at[j1]], ssem1, add=True)
        pltpu.make_async_copy(rows0d, deg_acc.at[sidx.at[0]], ssem0).wait()
        return carry
    lax.fori_loop(0, NPAIR, _dbody, 0)
    pltpu.make_async_copy(rows1d, deg_acc.at[sidx.at[0]], ssem1).wait()
    plsc.subcore_barrier()

    pltpu.sync_copy(deg_acc.at[pl.ds(s * DPT, DPT)], deg16)
    czero = jnp.zeros((16,), jnp.int32)

    def _dis(i, carry):
        ridx = lax.iota(jnp.int32, 16) + i * 16
        d = plsc.load_gather(deg16, [ridx, czero])
        m = d > 0.0
        d1 = jnp.where(m, d, 1.0)
        ii = plsc.bitcast(d1, jnp.int32)
        ii = 0x5F3759DF - jnp.right_shift(ii, 1)
        y = plsc.bitcast(ii, jnp.float32)
        for _it in range(3):
            y = y * (1.5 - 0.5 * d1 * y * y)
        disbuf[pl.ds(i * 16, 16)] = jnp.where(m, y, 0.0)
        return carry
    lax.fori_loop(0, DPT // 16, _dis, 0)
    pltpu.sync_copy(disbuf, dis_sh.at[pl.ds(s * DPT, DPT)])
    plsc.subcore_barrier()
    pltpu.sync_copy(dis_sh, disfull)

    wbase = c * (E2 // B) + s * WB_
    pltpu.sync_copy(srcb.at[pl.ds(wbase, WB_)], srcw)
    pltpu.sync_copy(dstb.at[pl.ds(wbase, WB_)], dstw)
    pltpu.sync_copy(ewb.at[pl.ds(wbase, WB_)], eww)

    def _w(j, carry):
        for k in range(B // 16):
            sv = srcw[j, pl.ds(k * 16, 16)]
            dv = dstw[j, pl.ds(k * 16, 16)]
            ev = eww[j, pl.ds(k * 16, 16)]
            a = plsc.load_gather(disfull, [sv])
            bb = plsc.load_gather(disfull, [dv])
            wout[j, pl.ds(k * 16, 16)] = -(a * ev * bb)
        return carry
    lax.fori_loop(0, WB_, _w, 0)
    pltpu.sync_copy(wout, w_out.at[pl.ds(wbase, WB_)])


BN = 1000
NBLK = N // BN


def _dot(a, b):
    return lax.dot_general(a, b, (((1,), (0,)), ((), ())),
                           preferred_element_type=jnp.float32)


def _tc1_body(x, h, p1xl, p1xh, p2xl, p2xh, p1hl, p1hh, p2hl, p2hh,
              ax, a1, a2, bh, b1, b2, bias, z_ref, ht1_ref, hrs_ref):
    a1v = a1[...]
    a2v = a2[...]
    b1v = b1[...]
    b2v = b2[...]
    accx = (_dot(x[...], ax[...])
            + _dot(p1xl[...], a1v[:DH]) + _dot(p1xh[...], a1v[DH:])
            + _dot(p2xl[...], a2v[:DH]) + _dot(p2xh[...], a2v[DH:]))
    acch = (_dot(h[...], bh[...])
            + _dot(p1hl[...], b1v[:DH]) + _dot(p1hh[...], b1v[DH:])
            + _dot(p2hl[...], b2v[:DH]) + _dot(p2hh[...], b2v[DH:]))
    pre = accx + bias[...]
    z = jax.nn.sigmoid(pre[:, :D] + acch[:, :D])
    r = jax.nn.sigmoid(pre[:, D:2 * D] + acch[:, D:])
    z_ref[...] = z
    ht1_ref[...] = pre[:, 2 * D:]
    hr = h[...] * r
    hrs_ref[0] = hr[:, :DH]
    hrs_ref[1] = hr[:, DH:]


def _tc2_body(hrl, hrh, p1l, p1h_, p2l, p2h_, ht1, z, h,
              c0, c1, c2, bias, out_ref):
    c0v = c0[...]
    c1v = c1[...]
    c2v = c2[...]
    ht2 = (_dot(hrl[...], c0v[:DH]) + _dot(hrh[...], c0v[DH:])
           + _dot(p1l[...], c1v[:DH]) + _dot(p1h_[...], c1v[DH:])
           + _dot(p2l[...], c2v[:DH]) + _dot(p2h_[...], c2v[DH:])
           + bias[...])
    sacc = ht1[...] + ht2
    htil = jnp.tanh(sacc) + sacc
    zv = z[...]
    out_ref[...] = zv * h[...] + (1.0 - zv) * htil


def _rb(i):
    return (i, 0)


def _hb(i):
    return (i + NBLK, 0)


def _full(i):
    return (0, 0)


def _tc1(X, H, P1x, P2x, P1h, P2h, Ax, A1, A2, Bh, B1, B2, bias):
    return pl.pallas_call(
        _tc1_body,
        grid=(NBLK,),
        in_specs=[
            pl.BlockSpec((BN, D), _rb),
            pl.BlockSpec((BN, D), _rb),
            pl.BlockSpec((BN, DH), _rb), pl.BlockSpec((BN, DH), _hb),
            pl.BlockSpec((BN, DH), _rb), pl.BlockSpec((BN, DH), _hb),
            pl.BlockSpec((BN, DH), _rb), pl.BlockSpec((BN, DH), _hb),
            pl.BlockSpec((BN, DH), _rb), pl.BlockSpec((BN, DH), _hb),
            pl.BlockSpec((D, 3 * D), _full), pl.BlockSpec((D, 3 * D), _full),
            pl.BlockSpec((D, 3 * D), _full),
            pl.BlockSpec((D, 2 * D), _full), pl.BlockSpec((D, 2 * D), _full),
            pl.BlockSpec((D, 2 * D), _full),
            pl.BlockSpec((1, 3 * D), _full),
        ],
        out_specs=[
            pl.BlockSpec((BN, D), _rb),
            pl.BlockSpec((BN, D), _rb),
            pl.BlockSpec((2, BN, DH), lambda i: (0, i, 0)),
        ],
        out_shape=[
            jax.ShapeDtypeStruct((N, D), jnp.float32),
            jax.ShapeDtypeStruct((N, D), jnp.float32),
            jax.ShapeDtypeStruct((2, N, DH), jnp.float32),
        ],
    )(X, H, P1x, P1x, P2x, P2x, P1h, P1h, P2h, P2h,
      Ax, A1, A2, Bh, B1, B2, bias)


def _tc2(HRs, P1hr, P2hr, Ht1, Z, H, C0, C1, C2, bias):
    return pl.pallas_call(
        _tc2_body,
        grid=(NBLK,),
        in_specs=[
            pl.BlockSpec((BN, DH), _rb), pl.BlockSpec((BN, DH), _hb),
            pl.BlockSpec((BN, DH), _rb), pl.BlockSpec((BN, DH), _hb),
            pl.BlockSpec((BN, DH), _rb), pl.BlockSpec((BN, DH), _hb),
            pl.BlockSpec((BN, D), _rb),
            pl.BlockSpec((BN, D), _rb),
            pl.BlockSpec((BN, D), _rb),
            pl.BlockSpec((D, D), _full), pl.BlockSpec((D, D), _full),
            pl.BlockSpec((D, D), _full),
            pl.BlockSpec((1, D), _full),
        ],
        out_specs=pl.BlockSpec((BN, D), _rb),
        out_shape=jax.ShapeDtypeStruct((N, D), jnp.float32),
    )(HRs, HRs, P1hr, P1hr, P2hr, P2hr, Ht1, Z, H, C0, C1, C2, bias)


def kernel(X, edge_index, edge_weight, H,
           W_xz, b_xz, W_hz, b_hz, W_xr, b_xr, W_hr, b_hr,
           W_xh, b_xh, W_hh, b_hh):
    src = edge_index[0]
    dst = edge_index[1]
    srcb = src.reshape(NBALL, B)
    dstb = dst.reshape(NBALL, B)
    ewb = edge_weight.reshape(NBALL, B)

    wb = _edge_prep(srcb, dstb, ewb)

    Xs = jnp.concatenate([X[:, :DH], X[:, DH:]], axis=0)
    Hs = jnp.concatenate([H[:, :DH], H[:, DH:]], axis=0)
    P1x, P2x, P1h, P2h = _quad_prop(Xs, Hs, srcb, dstb, wb)

    Ax = jnp.concatenate([W_xz[0] - W_xz[2], W_xr[0] - W_xr[2],
                          W_xh[0] - W_xh[2]], axis=1)
    A1 = jnp.concatenate([W_xz[1], W_xr[1], W_xh[1]], axis=1)
    A2 = jnp.concatenate([2.0 * W_xz[2], 2.0 * W_xr[2], 2.0 * W_xh[2]], axis=1)
    Bh = jnp.concatenate([W_hz[0] - W_hz[2], W_hr[0] - W_hr[2]], axis=1)
    B1 = jnp.concatenate([W_hz[1], W_hr[1]], axis=1)
    B2 = jnp.concatenate([2.0 * W_hz[2], 2.0 * W_hr[2]], axis=1)
    bias = jnp.concatenate([b_xz + b_hz, b_xr + b_hr, b_xh])[None, :]

    Z, Ht1, HRs3 = _tc1(X, H, P1x, P2x, P1h, P2h, Ax, A1, A2, Bh, B1, B2, bias)
    HRs = HRs3.reshape(2 * N, DH)

    P1hr, P2hr = _pair_prop(HRs, srcb, dstb, wb)

    C0 = W_hh[0] - W_hh[2]
    C1 = W_hh[1]
    C2 = 2.0 * W_hh[2]
    return _tc2(HRs, P1hr, P2hr, Ht1, Z, H, C0, C1, C2, b_hh[None, :])
